# baseline (speedup 1.0000x reference)
import jax
import jax.numpy as jnp
from jax.experimental import pallas as pl

_BM = 1024
_KF = 16


def _fused_body(s_ref, g_ref, c_ref, v_ref, o_ref):
    bm = s_ref.shape[0]
    p = jnp.concatenate([s_ref[...], jnp.ones((bm, 1), jnp.float32)], axis=1)
    pt = jnp.concatenate([p, p, p, p], axis=1)
    pr = jnp.concatenate([p[:, 0:1], p[:, 0:1], p[:, 0:1], p[:, 0:1],
                          p[:, 1:2], p[:, 1:2], p[:, 1:2], p[:, 1:2],
                          p[:, 2:3], p[:, 2:3], p[:, 2:3], p[:, 2:3],
                          p[:, 3:4], p[:, 3:4], p[:, 3:4], p[:, 3:4]],
                         axis=1)
    f16 = pr * pt
    s = jnp.dot(f16, g_ref[...], preferred_element_type=jnp.float32)
    s = s + c_ref[0:1, :]
    w = jnp.exp(s)
    o_ref[...] = jnp.dot(w, v_ref[...], preferred_element_type=jnp.float32)


def kernel(means, values, covariances, conics, opacities, samples):
    del covariances
    M = samples.shape[0]
    N = means.shape[0]
    C = values.shape[1]

    A11, A12, A13, A22, A23, A33 = [conics[:, i] for i in range(6)]
    mx, my, mz = means[:, 0], means[:, 1], means[:, 2]
    bx = A11 * mx + A12 * my + A13 * mz
    by = A12 * mx + A22 * my + A23 * mz
    bz = A13 * mx + A23 * my + A33 * mz
    c = mx * bx + my * by + mz * bz
    hbx, hby, hbz = 0.5 * bx, 0.5 * by, 0.5 * bz
    zn = jnp.zeros((N,), jnp.float32)
    g16 = [-0.5 * A11, -0.5 * A12, -0.5 * A13, hbx,
           -0.5 * A12, -0.5 * A22, -0.5 * A23, hby,
           -0.5 * A13, -0.5 * A23, -0.5 * A33, hbz,
           hbx, hby, hbz, zn]
    g_mat = jnp.stack(g16, axis=1).T
    c_mat = jnp.broadcast_to((-0.5 * c)[None, :], (8, N))
    v_mat = opacities * values

    out = pl.pallas_call(
        _fused_body,
        grid=(M // _BM,),
        in_specs=[
            pl.BlockSpec((_BM, 3), lambda i: (i, 0)),
            pl.BlockSpec((_KF, N), lambda i: (0, 0)),
            pl.BlockSpec((8, N), lambda i: (0, 0)),
            pl.BlockSpec((N, C), lambda i: (0, 0)),
        ],
        out_specs=pl.BlockSpec((_BM, C), lambda i: (i, 0)),
        out_shape=jax.ShapeDtypeStruct((M, C), jnp.float32),
    )(samples, g_mat, c_mat, v_mat)
    return out


# speedup vs baseline: 1.1994x; 1.1994x over previous
import jax
import jax.numpy as jnp
from jax.experimental import pallas as pl

_BM = 1024
_KF = 32


def _fused_body(f_ref, g_ref, c_ref, v_ref, o_ref):
    s = jnp.dot(f_ref[...], g_ref[...], preferred_element_type=jnp.float32)
    s = s + c_ref[0:1, :]
    w = jnp.exp(s)
    o_ref[...] = jnp.dot(w, v_ref[...], preferred_element_type=jnp.float32)


def _split_hi_lo(x):
    hi = x.astype(jnp.bfloat16)
    lo = (x - hi.astype(jnp.float32)).astype(jnp.bfloat16)
    return hi, lo


def kernel(means, values, covariances, conics, opacities, samples):
    del covariances
    M = samples.shape[0]
    N = means.shape[0]
    C = values.shape[1]

    A11, A12, A13, A22, A23, A33 = [conics[:, i] for i in range(6)]
    mx, my, mz = means[:, 0], means[:, 1], means[:, 2]
    bx = A11 * mx + A12 * my + A13 * mz
    by = A12 * mx + A22 * my + A23 * mz
    bz = A13 * mx + A23 * my + A33 * mz
    c = mx * bx + my * by + mz * bz
    g9 = jnp.stack([-0.5 * A11, -0.5 * A12, -0.5 * A13,
                    -0.5 * A22, -0.5 * A23, -0.5 * A33,
                    bx, by, bz], axis=0)
    c_mat = jnp.broadcast_to((-0.5 * c)[None, :], (8, N))

    sx, sy, sz = samples[:, 0], samples[:, 1], samples[:, 2]
    f9 = jnp.stack([sx * sx, 2.0 * sx * sy, 2.0 * sx * sz,
                    sy * sy, 2.0 * sy * sz, sz * sz,
                    sx, sy, sz], axis=1)

    f_hi, f_lo = _split_hi_lo(f9)
    g_hi, g_lo = _split_hi_lo(g9)
    zf = jnp.zeros((M, 5), jnp.bfloat16)
    zg = jnp.zeros((5, N), jnp.bfloat16)
    f_mat = jnp.concatenate([f_hi, f_hi, f_lo, zf], axis=1)
    g_mat = jnp.concatenate([g_hi, g_lo, g_hi, zg], axis=0)

    v_mat = opacities * values

    out = pl.pallas_call(
        _fused_body,
        grid=(M // _BM,),
        in_specs=[
            pl.BlockSpec((_BM, _KF), lambda i: (i, 0)),
            pl.BlockSpec((_KF, N), lambda i: (0, 0)),
            pl.BlockSpec((8, N), lambda i: (0, 0)),
            pl.BlockSpec((N, C), lambda i: (0, 0)),
        ],
        out_specs=pl.BlockSpec((_BM, C), lambda i: (i, 0)),
        out_shape=jax.ShapeDtypeStruct((M, C), jnp.float32),
    )(f_mat, g_mat, c_mat, v_mat)
    return out


# speedup vs baseline: 1.2105x; 1.0093x over previous
import jax
import jax.numpy as jnp
from jax.experimental import pallas as pl

_BM = 2048
_KF = 32


def _fused_body(f_ref, g_ref, c_ref, v_ref, o_ref):
    s = jnp.dot(f_ref[...], g_ref[...], preferred_element_type=jnp.float32)
    s = s + c_ref[0:1, :]
    w = jnp.exp(s)
    o_ref[...] = jnp.dot(w, v_ref[...], preferred_element_type=jnp.float32)


def _split_hi_lo(x):
    hi = x.astype(jnp.bfloat16)
    lo = (x - hi.astype(jnp.float32)).astype(jnp.bfloat16)
    return hi, lo


def kernel(means, values, covariances, conics, opacities, samples):
    del covariances
    M = samples.shape[0]
    N = means.shape[0]
    C = values.shape[1]

    A11, A12, A13, A22, A23, A33 = [conics[:, i] for i in range(6)]
    mx, my, mz = means[:, 0], means[:, 1], means[:, 2]
    bx = A11 * mx + A12 * my + A13 * mz
    by = A12 * mx + A22 * my + A23 * mz
    bz = A13 * mx + A23 * my + A33 * mz
    c = mx * bx + my * by + mz * bz
    g9 = jnp.stack([-0.5 * A11, -0.5 * A12, -0.5 * A13,
                    -0.5 * A22, -0.5 * A23, -0.5 * A33,
                    bx, by, bz], axis=0)
    c_mat = jnp.broadcast_to((-0.5 * c)[None, :], (8, N))

    sx, sy, sz = samples[:, 0], samples[:, 1], samples[:, 2]
    f9 = jnp.stack([sx * sx, 2.0 * sx * sy, 2.0 * sx * sz,
                    sy * sy, 2.0 * sy * sz, sz * sz,
                    sx, sy, sz], axis=1)

    f_hi, f_lo = _split_hi_lo(f9)
    g_hi, g_lo = _split_hi_lo(g9)
    zf = jnp.zeros((M, 5), jnp.bfloat16)
    zg = jnp.zeros((5, N), jnp.bfloat16)
    f_mat = jnp.concatenate([f_hi, f_hi, f_lo, zf], axis=1)
    g_mat = jnp.concatenate([g_hi, g_lo, g_hi, zg], axis=0)

    v_mat = opacities * values

    out = pl.pallas_call(
        _fused_body,
        grid=(M // _BM,),
        in_specs=[
            pl.BlockSpec((_BM, _KF), lambda i: (i, 0)),
            pl.BlockSpec((_KF, N), lambda i: (0, 0)),
            pl.BlockSpec((8, N), lambda i: (0, 0)),
            pl.BlockSpec((N, C), lambda i: (0, 0)),
        ],
        out_specs=pl.BlockSpec((_BM, C), lambda i: (i, 0)),
        out_shape=jax.ShapeDtypeStruct((M, C), jnp.float32),
    )(f_mat, g_mat, c_mat, v_mat)
    return out
